# Initial kernel scaffold; baseline (speedup 1.0000x reference)
import jax
import jax.numpy as jnp
from jax import lax
from jax.experimental import pallas as pl
from jax.experimental.pallas import tpu as pltpu
from jax.experimental.pallas import tpu_sc as plsc

B = 16384
F = 26
V = 100000
D = 32
ND = F * D
H1 = 256
H2 = 128

NW = 32
ROWS = B * F
RPW = ROWS // NW
CH = 1024
NCH = RPW // CH
GSZ = 128
NSUB = CH // GSZ


def _gather_body(x_hbm, tab_hbm, out_hbm, xbuf, idxbuf, rowbuf, sem):
    wid = lax.axis_index("s") * 2 + lax.axis_index("c")
    base = wid * RPW

    def chunk(c, carry):
        rb = base + c * CH
        pltpu.sync_copy(x_hbm.at[pl.ds(rb, CH)], xbuf)

        def idx16(t, carry2):
            off = pl.multiple_of(t * 16, 16)
            lane = lax.broadcasted_iota(jnp.int32, (16,), 0)
            g = rb + off + lane
            f = lax.rem(g, F)
            idxbuf[pl.ds(off, 16)] = xbuf[pl.ds(off, 16)] + f * V
            return carry2

        lax.fori_loop(0, CH // 16, idx16, 0)
        cps = [
            pltpu.async_copy(
                tab_hbm.at[idxbuf.at[pl.ds(j * GSZ, GSZ)]],
                rowbuf.at[pl.ds(j * GSZ, GSZ)],
                sem,
            )
            for j in range(NSUB)
        ]
        for cp in cps:
            cp.wait()
        pltpu.sync_copy(rowbuf, out_hbm.at[pl.ds(rb, CH)])
        return carry

    lax.fori_loop(0, NCH, chunk, 0)


_gather = pl.kernel(
    _gather_body,
    out_type=jax.ShapeDtypeStruct((ROWS, D), jnp.float32),
    mesh=plsc.VectorSubcoreMesh(core_axis_name="c", subcore_axis_name="s"),
    scratch_types=[
        pltpu.VMEM((CH,), jnp.int32),
        pltpu.VMEM((CH,), jnp.int32),
        pltpu.VMEM((CH, D), jnp.float32),
        pltpu.SemaphoreType.DMA,
    ],
)


BK = 1024


def _mlp_body(e_ref, w1_ref, b1_ref, w2_ref, b2_ref, w3_ref, b3_ref, o_ref):
    e = e_ref[...]
    h = jnp.dot(e, w1_ref[...], preferred_element_type=jnp.float32)
    h = jnp.maximum(h + b1_ref[...][None, :], 0.0)
    h = jnp.dot(h, w2_ref[...], preferred_element_type=jnp.float32)
    h = jnp.maximum(h + b2_ref[...][None, :], 0.0)
    z = jnp.sum(h * w3_ref[...][None, :], axis=1) + b3_ref[...]
    o_ref[...] = 1.0 / (1.0 + jnp.exp(-z))


_mlp = pl.pallas_call(
    _mlp_body,
    grid=(B // BK,),
    in_specs=[
        pl.BlockSpec((BK, ND), lambda i: (i, 0)),
        pl.BlockSpec((ND, H1), lambda i: (0, 0)),
        pl.BlockSpec((H1,), lambda i: (0,)),
        pl.BlockSpec((H1, H2), lambda i: (0, 0)),
        pl.BlockSpec((H2,), lambda i: (0,)),
        pl.BlockSpec((H2,), lambda i: (0,)),
        pl.BlockSpec((1,), lambda i: (0,)),
    ],
    out_specs=pl.BlockSpec((BK,), lambda i: (i,)),
    out_shape=jax.ShapeDtypeStruct((B,), jnp.float32),
    compiler_params=pltpu.CompilerParams(
        dimension_semantics=("parallel",),
    ),
)


def kernel(x, tables, W1, b1, W2, b2, W3, b3):
    xf = x.reshape(ROWS).astype(jnp.int32)
    tflat = tables.reshape(F * V, D)
    rows = _gather(xf, tflat)
    emb = rows.reshape(B, ND)
    return _mlp(emb, W1, b1, W2, b2, W3.reshape(H2), b3)


# speedup vs baseline: 8.0005x; 8.0005x over previous
import jax
import jax.numpy as jnp
from jax import lax
from jax.experimental import pallas as pl
from jax.experimental.pallas import tpu as pltpu
from jax.experimental.pallas import tpu_sc as plsc

B = 16384
F = 26
V = 100000
D = 32
ND = F * D
H1 = 256
H2 = 128

NW = 32
ROWS = B * F
RPW = ROWS // NW
CH = 1024
NCH = RPW // CH
GSZ = 128
NSUB = CH // GSZ


def _gather_body(x_hbm, tab_hbm, out_hbm, xbuf, idxbuf, rowbuf, sem):
    wid = lax.axis_index("s") * 2 + lax.axis_index("c")
    base = wid * RPW

    def chunk(c, carry):
        rb = base + c * CH
        pltpu.sync_copy(x_hbm.at[pl.ds(rb, CH)], xbuf)

        def idx16(t, carry2):
            off = pl.multiple_of(t * 16, 16)
            lane = lax.broadcasted_iota(jnp.int32, (16,), 0)
            g = rb + off + lane
            f = lax.rem(g, F)
            idxbuf[pl.ds(off, 16)] = xbuf[pl.ds(off, 16)] + f * V
            return carry2

        lax.fori_loop(0, CH // 16, idx16, 0)
        cps = [
            pltpu.async_copy(
                tab_hbm.at[idxbuf.at[pl.ds(j * GSZ, GSZ)]],
                rowbuf.at[pl.ds(j * GSZ, GSZ)],
                sem,
            )
            for j in range(NSUB)
        ]
        for cp in cps:
            cp.wait()
        pltpu.sync_copy(rowbuf, out_hbm.at[pl.ds(rb, CH)])
        return carry

    lax.fori_loop(0, NCH, chunk, 0)


_gather_cache = []


def _gather(xf, tflat):
    if not _gather_cache:
        _gather_cache.append(pl.kernel(
            _gather_body,
            out_type=jax.ShapeDtypeStruct((ROWS, D), jnp.float32),
            mesh=plsc.VectorSubcoreMesh(core_axis_name="c", subcore_axis_name="s"),
            scratch_types=[
                pltpu.VMEM((CH,), jnp.int32),
                pltpu.VMEM((CH,), jnp.int32),
                pltpu.VMEM((CH, D), jnp.float32),
                pltpu.SemaphoreType.DMA,
            ],
            compiler_params=pltpu.CompilerParams(use_tc_tiling_on_sc=False),
        ))
    return _gather_cache[0](xf, tflat)


BK = 1024


def _mlp_body(e_ref, w1_ref, b1_ref, w2_ref, b2_ref, w3_ref, b3_ref, o_ref):
    e = e_ref[...]
    h = jnp.dot(e, w1_ref[...], preferred_element_type=jnp.float32)
    h = jnp.maximum(h + b1_ref[...][None, :], 0.0)
    h = jnp.dot(h, w2_ref[...], preferred_element_type=jnp.float32)
    h = jnp.maximum(h + b2_ref[...][None, :], 0.0)
    z = jnp.sum(h * w3_ref[...][None, :], axis=1) + b3_ref[...]
    o_ref[...] = 1.0 / (1.0 + jnp.exp(-z))


_mlp = pl.pallas_call(
    _mlp_body,
    grid=(B // BK,),
    in_specs=[
        pl.BlockSpec((BK, ND), lambda i: (i, 0)),
        pl.BlockSpec((ND, H1), lambda i: (0, 0)),
        pl.BlockSpec((H1,), lambda i: (0,)),
        pl.BlockSpec((H1, H2), lambda i: (0, 0)),
        pl.BlockSpec((H2,), lambda i: (0,)),
        pl.BlockSpec((H2,), lambda i: (0,)),
        pl.BlockSpec((1,), lambda i: (0,)),
    ],
    out_specs=pl.BlockSpec((BK,), lambda i: (i,)),
    out_shape=jax.ShapeDtypeStruct((B,), jnp.float32),
    compiler_params=pltpu.CompilerParams(
        dimension_semantics=("parallel",),
    ),
)


def kernel(x, tables, W1, b1, W2, b2, W3, b3):
    xf = x.reshape(ROWS).astype(jnp.int32)
    tflat = tables.reshape(F * V, D)
    rows = _gather(xf, tflat)
    emb = rows.reshape(B, ND)
    return _mlp(emb, W1, b1, W2, b2, W3.reshape(H2), b3)


# speedup vs baseline: 29.5653x; 3.6954x over previous
import jax
import jax.numpy as jnp
from jax import lax
from jax.experimental import pallas as pl
from jax.experimental.pallas import tpu as pltpu
from jax.experimental.pallas import tpu_sc as plsc

B = 16384
F = 26
V = 100000
D = 32
ND = F * D
H1 = 256
H2 = 128

NW = 32
HB = B // 2


def _gather_body(xt_hbm, tab_hbm, out_hbm, xbuf, obuf, rowbuf, sem):
    w = lax.axis_index("s") * 2 + lax.axis_index("c")

    def field(f, carry):
        r = f * D + w
        pltpu.sync_copy(tab_hbm.at[r], rowbuf)

        def half(h, carry2):
            hb = pl.multiple_of(h * HB, HB)
            pltpu.sync_copy(xt_hbm.at[f, pl.ds(hb, HB)], xbuf)

            def g128(i, carry3):
                base = pl.multiple_of(i * 128, 128)
                for u in range(8):
                    off = base + u * 16
                    idx = xbuf[pl.ds(off, 16)]
                    obuf[pl.ds(off, 16)] = plsc.load_gather(rowbuf, [idx])
                return carry3

            lax.fori_loop(0, HB // 128, g128, 0)
            pltpu.sync_copy(obuf, out_hbm.at[r, pl.ds(hb, HB)])
            return carry2

        lax.fori_loop(0, 2, half, 0)
        return carry

    lax.fori_loop(0, F, field, 0)


_gather_cache = []


def _gather(xt, tab):
    if not _gather_cache:
        _gather_cache.append(pl.kernel(
            _gather_body,
            out_type=jax.ShapeDtypeStruct((ND, B), jnp.float32),
            mesh=plsc.VectorSubcoreMesh(core_axis_name="c", subcore_axis_name="s"),
            scratch_types=[
                pltpu.VMEM((HB,), jnp.int32),
                pltpu.VMEM((HB,), jnp.float32),
                pltpu.VMEM((V,), jnp.float32),
                pltpu.SemaphoreType.DMA,
            ],
            compiler_params=pltpu.CompilerParams(
                use_tc_tiling_on_sc=True, needs_layout_passes=False),
        ))
    return _gather_cache[0](xt, tab)


BK = 1024


def _mlp_body(e_ref, w1t_ref, b1_ref, w2t_ref, b2_ref, w3_ref, b3_ref, o_ref):
    e = e_ref[...]
    h = jnp.dot(w1t_ref[...], e, preferred_element_type=jnp.float32)
    h = jnp.maximum(h + b1_ref[...][:, None], 0.0)
    h = jnp.dot(w2t_ref[...], h, preferred_element_type=jnp.float32)
    h = jnp.maximum(h + b2_ref[...][:, None], 0.0)
    z = jnp.sum(h * w3_ref[...][:, None], axis=0) + b3_ref[...]
    o_ref[...] = 1.0 / (1.0 + jnp.exp(-z))


_mlp = pl.pallas_call(
    _mlp_body,
    grid=(B // BK,),
    in_specs=[
        pl.BlockSpec((ND, BK), lambda i: (0, i)),
        pl.BlockSpec((H1, ND), lambda i: (0, 0)),
        pl.BlockSpec((H1,), lambda i: (0,)),
        pl.BlockSpec((H2, H1), lambda i: (0, 0)),
        pl.BlockSpec((H2,), lambda i: (0,)),
        pl.BlockSpec((H2,), lambda i: (0,)),
        pl.BlockSpec((1,), lambda i: (0,)),
    ],
    out_specs=pl.BlockSpec((BK,), lambda i: (i,)),
    out_shape=jax.ShapeDtypeStruct((B,), jnp.float32),
    compiler_params=pltpu.CompilerParams(
        dimension_semantics=("parallel",),
    ),
)


def kernel(x, tables, W1, b1, W2, b2, W3, b3):
    xt = x.astype(jnp.int32).T
    tab = tables.transpose(0, 2, 1).reshape(ND, V)
    embT = _gather(xt, tab)
    return _mlp(embT, W1.T, b1, W2.T, b2, W3.reshape(H2), b3)


# speedup vs baseline: 39.8644x; 1.3484x over previous
import jax
import jax.numpy as jnp
from jax import lax
from jax.experimental import pallas as pl
from jax.experimental.pallas import tpu as pltpu
from jax.experimental.pallas import tpu_sc as plsc

B = 16384
F = 26
V = 100000
D = 32
ND = F * D
H1 = 256
H2 = 128

NW = 32
QB = 4096
NQ = B // QB


def _gather_body(xt_hbm, tab_hbm, out_hbm, xq0, xq1, oq0, oq1, rowbuf,
                 sem_row, sem_x0, sem_x1, sem_o0, sem_o1):
    w = lax.axis_index("s") * 2 + lax.axis_index("c")
    xq = (xq0, xq1)
    sx = (sem_x0, sem_x1)
    oq = (oq0, oq1)
    so = (sem_o0, sem_o1)

    pltpu.async_copy(tab_hbm.at[w], rowbuf, sem_row)
    pltpu.async_copy(xt_hbm.at[0, pl.ds(0, QB)], xq0, sem_x0)

    def field(f, carry):
        r = f * D + w
        pltpu.make_async_copy(tab_hbm.at[r], rowbuf, sem_row).wait()
        pltpu.make_async_copy(xt_hbm.at[f, pl.ds(0, QB)], xq0, sem_x0).wait()

        odesc = {}
        for q in range(NQ):
            cur = q % 2
            if q + 1 < NQ:
                xdesc = pltpu.async_copy(
                    xt_hbm.at[f, pl.ds((q + 1) * QB, QB)], xq[(q + 1) % 2],
                    sx[(q + 1) % 2])
            if q >= 2:
                odesc[q - 2].wait()

            @plsc.parallel_loop(0, QB // 16, unroll=8)
            def g16(i):
                off = pl.multiple_of(i * 16, 16)
                idx = xq[cur][pl.ds(off, 16)]
                oq[cur][pl.ds(off, 16)] = plsc.load_gather(rowbuf, [idx])

            odesc[q] = pltpu.async_copy(
                oq[cur], out_hbm.at[r, pl.ds(q * QB, QB)], so[cur])
            if q + 1 < NQ:
                xdesc.wait()

        fn = jnp.minimum(f + 1, F - 1)
        pltpu.async_copy(tab_hbm.at[fn * D + w], rowbuf, sem_row)
        pltpu.async_copy(xt_hbm.at[fn, pl.ds(0, QB)], xq0, sem_x0)
        odesc[NQ - 2].wait()
        odesc[NQ - 1].wait()
        return carry

    lax.fori_loop(0, F, field, 0)
    pltpu.make_async_copy(tab_hbm.at[(F - 1) * D + w], rowbuf, sem_row).wait()
    pltpu.make_async_copy(xt_hbm.at[F - 1, pl.ds(0, QB)], xq0, sem_x0).wait()


_gather_cache = []


def _gather(xt, tab):
    if not _gather_cache:
        _gather_cache.append(pl.kernel(
            _gather_body,
            out_type=jax.ShapeDtypeStruct((ND, B), jnp.float32),
            mesh=plsc.VectorSubcoreMesh(core_axis_name="c", subcore_axis_name="s"),
            scratch_types=[
                pltpu.VMEM((QB,), jnp.int32),
                pltpu.VMEM((QB,), jnp.int32),
                pltpu.VMEM((QB,), jnp.float32),
                pltpu.VMEM((QB,), jnp.float32),
                pltpu.VMEM((V,), jnp.float32),
                pltpu.SemaphoreType.DMA,
                pltpu.SemaphoreType.DMA,
                pltpu.SemaphoreType.DMA,
                pltpu.SemaphoreType.DMA,
                pltpu.SemaphoreType.DMA,
            ],
            compiler_params=pltpu.CompilerParams(
                use_tc_tiling_on_sc=True, needs_layout_passes=False),
        ))
    return _gather_cache[0](xt, tab)


BK = 1024


def _mlp_body(e_ref, w1t_ref, b1_ref, w2t_ref, b2_ref, w3_ref, b3_ref, o_ref):
    e = e_ref[...]
    h = jnp.dot(w1t_ref[...], e, preferred_element_type=jnp.float32)
    h = jnp.maximum(h + b1_ref[...][:, None], 0.0)
    h = jnp.dot(w2t_ref[...], h, preferred_element_type=jnp.float32)
    h = jnp.maximum(h + b2_ref[...][:, None], 0.0)
    z = jnp.sum(h * w3_ref[...][:, None], axis=0) + b3_ref[...]
    o_ref[...] = 1.0 / (1.0 + jnp.exp(-z))


_mlp = pl.pallas_call(
    _mlp_body,
    grid=(B // BK,),
    in_specs=[
        pl.BlockSpec((ND, BK), lambda i: (0, i)),
        pl.BlockSpec((H1, ND), lambda i: (0, 0)),
        pl.BlockSpec((H1,), lambda i: (0,)),
        pl.BlockSpec((H2, H1), lambda i: (0, 0)),
        pl.BlockSpec((H2,), lambda i: (0,)),
        pl.BlockSpec((H2,), lambda i: (0,)),
        pl.BlockSpec((1,), lambda i: (0,)),
    ],
    out_specs=pl.BlockSpec((BK,), lambda i: (i,)),
    out_shape=jax.ShapeDtypeStruct((B,), jnp.float32),
    compiler_params=pltpu.CompilerParams(
        dimension_semantics=("parallel",),
    ),
)


def kernel(x, tables, W1, b1, W2, b2, W3, b3):
    xt = x.astype(jnp.int32).T
    tab = tables.transpose(0, 2, 1).reshape(ND, V)
    embT = _gather(xt, tab)
    return _mlp(embT, W1.T, b1, W2.T, b2, W3.reshape(H2), b3)
